# baseline (speedup 1.0000x reference)
import functools

import jax
import jax.numpy as jnp
from jax.experimental import pallas as pl
from jax.experimental.pallas import tpu as pltpu

N_POINTS = 20000
NUM_NODES = 512
POINT_LIMIT = 64
NUM_CORR = 256
FINE_TOPK = 3
OT_ITERS = 100

_BB = 32


def _lse(x, axis):
    m = jnp.max(x, axis=axis)
    return jnp.log(jnp.sum(jnp.exp(x - jnp.expand_dims(m, axis)), axis=axis)) + m


def _coarse_kernel(tf_ref, sf_ref, tm_ref, sm_ref, fidx_ref):
    Mn = tf_ref.shape[0]
    s1 = jax.lax.dot_general(tf_ref[...], sf_ref[...], (((1,), (1,)), ((), ())),
                             preferred_element_type=jnp.float32)
    d = jnp.maximum(2.0 - 2.0 * s1, 0.0)
    s = jnp.exp(-d)
    s = (s / jnp.sum(s, axis=1, keepdims=True)) * (s / jnp.sum(s, axis=0, keepdims=True))
    valid = (tm_ref[...] > 0.0) & (jnp.transpose(sm_ref[...]) > 0.0)
    s = jnp.where(valid, s, 0.0)
    flat = (jax.lax.broadcasted_iota(jnp.int32, (Mn, Mn), 0) * Mn
            + jax.lax.broadcasted_iota(jnp.int32, (Mn, Mn), 1)).astype(jnp.float32)
    kcol = jax.lax.broadcasted_iota(jnp.int32, (1, NUM_CORR), 1)

    def body(k, carry):
        s, acc = carry
        m1 = jnp.max(jnp.max(s, axis=1, keepdims=True), axis=0, keepdims=True)
        eq = s == m1
        fmin = jnp.min(jnp.min(jnp.where(eq, flat, 3.0e8), axis=1, keepdims=True),
                       axis=0, keepdims=True)
        s = jnp.where(eq & (flat == fmin), -1.0, s)
        return s, jnp.where(kcol == k, fmin, acc)

    _, acc = jax.lax.fori_loop(0, NUM_CORR, body,
                               (s, jnp.zeros((1, NUM_CORR), jnp.float32)))
    fidx_ref[...] = acc


def _run_coarse(tgt_node_f, src_node_f, tgt_nm, src_nm):
    M = NUM_NODES
    fidx = pl.pallas_call(
        _coarse_kernel,
        out_shape=jax.ShapeDtypeStruct((1, NUM_CORR), jnp.float32),
    )(tgt_node_f, src_node_f,
      tgt_nm.astype(jnp.float32).reshape(M, 1),
      src_nm.astype(jnp.float32).reshape(M, 1))
    flat_idx = fidx[0].astype(jnp.int32)
    return flat_idx // M, flat_idx % M


def _ot_kernel(scores_ref, rowm_ref, colm_ref, alpha_ref, tpts_ref, spts_ref,
               out_ref, cs_ref, tcp_ref, scp_ref):
    scores = scores_ref[...]
    rowm = rowm_ref[...]
    colm = colm_ref[...]
    alpha = alpha_ref[0, 0]
    b = scores.shape[0]
    m1 = scores.shape[1]

    ridx = jax.lax.broadcasted_iota(jnp.int32, (b, m1, m1), 1)
    cidx = jax.lax.broadcasted_iota(jnp.int32, (b, m1, m1), 2)
    is_pad = (ridx == m1 - 1) | (cidx == m1 - 1)
    padded = jnp.where(is_pad, alpha, scores)

    ridx2 = jax.lax.broadcasted_iota(jnp.int32, (b, m1), 1)
    prm = jnp.where(ridx2 == m1 - 1, 1.0, jnp.pad(rowm, ((0, 0), (0, 1))))
    pcm = jnp.where(ridx2 == m1 - 1, 1.0, jnp.pad(colm, ((0, 0), (0, 1))))
    sm = (prm[:, :, None] * pcm[:, None, :]) > 0.5
    padded = jnp.where(sm, padded, -1e5)

    nvr = jnp.maximum(jnp.sum(rowm, axis=1), 1.0)
    nvc = jnp.maximum(jnp.sum(colm, axis=1), 1.0)
    norm = -jnp.log(nvr + nvc)
    log_mu = jnp.where(ridx2 == m1 - 1, (jnp.log(nvc) + norm)[:, None], norm[:, None])
    log_nu = jnp.where(ridx2 == m1 - 1, (jnp.log(nvr) + norm)[:, None], norm[:, None])

    def step(i, uv):
        u, v = uv
        u = log_mu - _lse(padded + v[:, None, :], axis=2)
        v = log_nu - _lse(padded + u[:, :, None], axis=1)
        return (u, v)

    u0 = jnp.zeros((b, m1), jnp.float32)
    v0 = jnp.zeros((b, m1), jnp.float32)
    u, v = jax.lax.fori_loop(0, OT_ITERS, step, (u0, v0))
    out = padded + u[:, :, None] + v[:, None, :] - norm[:, None, None]
    out_ref[...] = out

    P = POINT_LIMIT
    fm = jnp.exp(out[:, :P, :P])
    fmask = (rowm[:, :, None] > 0.5) & (colm[:, None, :] > 0.5)
    fm = jnp.where(fmask, fm, 0.0)
    flat = (jax.lax.broadcasted_iota(jnp.int32, (b, P, P), 1) * P
            + jax.lax.broadcasted_iota(jnp.int32, (b, P, P), 2)).astype(jnp.float32)
    kio = jax.lax.broadcasted_iota(jnp.int32, (b, 1, P), 2).astype(jnp.float32)
    tio = jax.lax.broadcasted_iota(jnp.int32, (b, FINE_TOPK), 1)
    tio3 = jax.lax.broadcasted_iota(jnp.int32, (b, FINE_TOPK, 3), 1)
    cs = jnp.zeros((b, FINE_TOPK), jnp.float32)
    tcp = jnp.zeros((b, FINE_TOPK, 3), jnp.float32)
    scp = jnp.zeros((b, FINE_TOPK, 3), jnp.float32)
    for t in range(FINE_TOPK):
        m2 = jnp.max(jnp.max(fm, axis=2, keepdims=True), axis=1, keepdims=True)
        eq = fm == m2
        fmin = jnp.min(jnp.min(jnp.where(eq, flat, 3.0e8), axis=2, keepdims=True),
                       axis=1, keepdims=True)
        fm = jnp.where(eq & (flat == fmin), -1.0, fm)
        ri = jnp.floor(fmin * (1.0 / P))
        ci = fmin - ri * P
        tsel = jnp.sum(jnp.where(kio == ri[:, :, 0][:, :, None], tpts_ref[...], 0.0),
                       axis=2)
        ssel = jnp.sum(jnp.where(kio == ci[:, :, 0][:, :, None], spts_ref[...], 0.0),
                       axis=2)
        cs = jnp.where(tio == t, m2[:, :, 0], cs)
        tcp = jnp.where(tio3 == t, tsel[:, None, :], tcp)
        scp = jnp.where(tio3 == t, ssel[:, None, :], scp)
    cs_ref[...] = cs
    tcp_ref[...] = tcp
    scp_ref[...] = scp


def _run_ot(scores, row_masks, col_masks, alpha, tgt_ck_pts, src_ck_pts):
    B = scores.shape[0]
    scores65 = jnp.pad(scores, ((0, 0), (0, 1), (0, 1)))
    rowm = row_masks.astype(jnp.float32)
    colm = col_masks.astype(jnp.float32)
    alpha_arr = jnp.full((1, 1), alpha, jnp.float32)
    tpts = jnp.swapaxes(tgt_ck_pts, 1, 2)
    spts = jnp.swapaxes(src_ck_pts, 1, 2)
    grid = (B // _BB,)
    return pl.pallas_call(
        _ot_kernel,
        grid=grid,
        in_specs=[
            pl.BlockSpec((_BB, 65, 65), lambda i: (i, 0, 0)),
            pl.BlockSpec((_BB, 64), lambda i: (i, 0)),
            pl.BlockSpec((_BB, 64), lambda i: (i, 0)),
            pl.BlockSpec(memory_space=pltpu.SMEM),
            pl.BlockSpec((_BB, 3, 64), lambda i: (i, 0, 0)),
            pl.BlockSpec((_BB, 3, 64), lambda i: (i, 0, 0)),
        ],
        out_specs=[
            pl.BlockSpec((_BB, 65, 65), lambda i: (i, 0, 0)),
            pl.BlockSpec((_BB, FINE_TOPK), lambda i: (i, 0)),
            pl.BlockSpec((_BB, FINE_TOPK, 3), lambda i: (i, 0, 0)),
            pl.BlockSpec((_BB, FINE_TOPK, 3), lambda i: (i, 0, 0)),
        ],
        out_shape=[
            jax.ShapeDtypeStruct((B, 65, 65), jnp.float32),
            jax.ShapeDtypeStruct((B, FINE_TOPK), jnp.float32),
            jax.ShapeDtypeStruct((B, FINE_TOPK, 3), jnp.float32),
            jax.ShapeDtypeStruct((B, FINE_TOPK, 3), jnp.float32),
        ],
        compiler_params=pltpu.CompilerParams(
            dimension_semantics=("parallel",),
        ),
    )(scores65, rowm, colm, alpha_arr, tpts, spts)


_NPAD = 20480
_PCH = 2048
_INF = 3.0e38


def _partition_kernel(pts_ref, nodes_ref, knn_ref, knnx_ref, cnt_ref, key_ref,
                      parg_ref):
    M = nodes_ref.shape[1]
    nx = nodes_ref[0, :, 0:1]
    ny = nodes_ref[0, :, 1:2]
    nz = nodes_ref[0, :, 2:3]
    cnt_ref[0, :, :] = jnp.zeros((M, 1), jnp.float32)

    def chunk_body(c, _):
        base = c * _PCH
        px = pts_ref[0, 0:1, pl.ds(base, _PCH)]
        py = pts_ref[0, 1:2, pl.ds(base, _PCH)]
        pz = pts_ref[0, 2:3, pl.ds(base, _PCH)]
        dx = nx - px
        dy = ny - py
        dz = nz - pz
        sq = (dx * dx + dy * dy) + dz * dz
        jglob = (jax.lax.broadcasted_iota(jnp.int32, (M, _PCH), 1)
                 + base).astype(jnp.float32)
        valid = jglob < float(N_POINTS)
        sq = jnp.where(valid, sq, _INF)
        key_ref[:, pl.ds(base, _PCH)] = sq
        colmin = jnp.min(sq, axis=0, keepdims=True)
        m_iota = jax.lax.broadcasted_iota(jnp.int32, (M, _PCH), 0).astype(jnp.float32)
        colarg = jnp.min(jnp.where(sq == colmin, m_iota, 1e9), axis=0,
                         keepdims=True)
        contrib = jnp.where((m_iota == colarg) & valid, 1.0, 0.0)
        cnt_ref[0, :, :] = cnt_ref[0, :, :] + jnp.sum(contrib, axis=1, keepdims=True)
        parg_ref[0:1, pl.ds(base, _PCH)] = colarg
        return 0

    jax.lax.fori_loop(0, _NPAD // _PCH, chunk_body, 0)

    kcol = jax.lax.broadcasted_iota(jnp.int32, (M, POINT_LIMIT), 1)
    nch = _NPAD // _PCH
    jio0 = jax.lax.broadcasted_iota(jnp.int32, (M, _PCH), 1).astype(jnp.float32)

    def ext_body(k, carry):
        kp, jp, acc = carry
        gk = jnp.full((M, 1), _INF, jnp.float32)
        gj = jnp.full((M, 1), _INF, jnp.float32)
        for c in range(nch):
            keys = key_ref[:, pl.ds(c * _PCH, _PCH)]
            jio = jio0 + float(c * _PCH)
            pred = (keys > kp) | ((keys == kp) & (jio > jp))
            kc = jnp.where(pred, keys, _INF)
            ck = jnp.min(kc, axis=1, keepdims=True)
            cj = jnp.min(jnp.where(kc == ck, jio, _INF), axis=1, keepdims=True)
            better = (ck < gk) | ((ck == gk) & (cj < gj))
            gk = jnp.where(better, ck, gk)
            gj = jnp.where(better, cj, gj)
        return gk, gj, jnp.where(kcol == k, gj, acc)

    _, _, acc = jax.lax.fori_loop(
        0, POINT_LIMIT, ext_body,
        (jnp.full((M, 1), -1.0, jnp.float32), jnp.full((M, 1), -1.0, jnp.float32),
         jnp.zeros((M, POINT_LIMIT), jnp.float32)))
    knn_ref[0, :, :] = acc

    G = _NPAD // 128

    def table_gl(row):
        return row.reshape(G, 128)

    tbl = jnp.concatenate(
        [table_gl(pts_ref[0, 0:1, :]), table_gl(pts_ref[0, 1:2, :]),
         table_gl(pts_ref[0, 2:3, :]), table_gl(parg_ref[0:1, :])], axis=0)
    tbl_hi = tbl.astype(jnp.bfloat16)
    tbl_mid = (tbl - tbl_hi.astype(jnp.float32)).astype(jnp.bfloat16)
    tbl_lo = (tbl - tbl_hi.astype(jnp.float32)
              - tbl_mid.astype(jnp.float32)).astype(jnp.bfloat16)
    acc_t = jnp.transpose(acc)
    dn = (((1,), (0,)), ((), ()))
    lio = jax.lax.broadcasted_iota(jnp.int32, (128, M), 0).astype(jnp.float32)
    gio = jax.lax.broadcasted_iota(jnp.int32, (G, M), 0).astype(jnp.float32)
    for k in range(POINT_LIMIT):
        row = jax.lax.slice(acc_t, (k, 0), (k + 1, M))
        hi = jnp.floor(row * (1.0 / 128.0))
        lo = row - hi * 128.0
        oh_lo = (lio == lo).astype(jnp.bfloat16)
        tmp = (jax.lax.dot_general(tbl_hi, oh_lo, dn,
                                   preferred_element_type=jnp.float32)
               + jax.lax.dot_general(tbl_mid, oh_lo, dn,
                                     preferred_element_type=jnp.float32)
               + jax.lax.dot_general(tbl_lo, oh_lo, dn,
                                     preferred_element_type=jnp.float32))
        oh_hi = jnp.where(gio == hi, 1.0, 0.0)
        for col in range(4):
            piece = jax.lax.slice(tmp, (col * G, 0), ((col + 1) * G, M))
            val = jnp.sum(piece * oh_hi, axis=0, keepdims=True)
            knnx_ref[0, col, k:k + 1, :] = val


def _run_partition(src_pts, tgt_pts, nodes_src, nodes_tgt):
    def prep(p):
        return jnp.pad(p, ((0, _NPAD - N_POINTS), (0, 0))).T
    pts2 = jnp.stack([prep(src_pts), prep(tgt_pts)])
    nodes2 = jnp.stack([nodes_src, nodes_tgt])
    M = NUM_NODES
    knn_f, knnx, cnt = pl.pallas_call(
        _partition_kernel,
        grid=(2,),
        in_specs=[
            pl.BlockSpec((1, 3, _NPAD), lambda i: (i, 0, 0)),
            pl.BlockSpec((1, M, 3), lambda i: (i, 0, 0)),
        ],
        out_specs=[
            pl.BlockSpec((1, M, POINT_LIMIT), lambda i: (i, 0, 0)),
            pl.BlockSpec((1, 4, POINT_LIMIT, M), lambda i: (i, 0, 0, 0)),
            pl.BlockSpec((1, M, 1), lambda i: (i, 0, 0)),
        ],
        out_shape=[
            jax.ShapeDtypeStruct((2, M, POINT_LIMIT), jnp.float32),
            jax.ShapeDtypeStruct((2, 4, POINT_LIMIT, M), jnp.float32),
            jax.ShapeDtypeStruct((2, M, 1), jnp.float32),
        ],
        scratch_shapes=[pltpu.VMEM((M, _NPAD), jnp.float32),
                        pltpu.VMEM((1, _NPAD), jnp.float32)],
        compiler_params=pltpu.CompilerParams(
            dimension_semantics=("parallel",),
        ),
    )(pts2, nodes2)
    knn_idx = knn_f.astype(jnp.int32)
    node_masks = cnt[:, :, 0] > 0
    return knn_idx, jnp.swapaxes(knnx, 2, 3), node_masks


def _finish_partition(knn_idx, knnx, node_masks):
    knn_node = knnx[3].astype(jnp.int32)
    knn_masks = (knn_node == jnp.arange(NUM_NODES)[:, None]) & node_masks[:, None]
    knn_idx = jnp.where(knn_masks, knn_idx, N_POINTS)
    knn_pts = jnp.where(knn_masks[:, :, None],
                        jnp.stack([knnx[0], knnx[1], knnx[2]], axis=-1), 0.0)
    return node_masks, knn_idx, knn_masks, knn_pts


def kernel(src_pcd, tgt_pcd, src_feats, tgt_feats, src_normals, tgt_normals,
           rot, trans, src_raw_pcd, W_node, W_point, W_coarse, b_coarse,
           W_fine, b_fine, alpha):
    src_pts, tgt_pts = src_pcd, tgt_pcd
    stride = src_pts.shape[0] // NUM_NODES
    node_idx = jnp.arange(NUM_NODES) * stride
    src_node_xyz = src_pts[node_idx]
    tgt_node_xyz = tgt_pts[node_idx]
    src_node_f = src_node_xyz @ W_node
    tgt_node_f = tgt_node_xyz @ W_node

    def l2n(x):
        return x / (jnp.linalg.norm(x, axis=1, keepdims=True) + 1e-12)

    src_node_f = l2n(src_node_f @ W_coarse + b_coarse)
    tgt_node_f = l2n(tgt_node_f @ W_coarse + b_coarse)

    knn2, knnx2, nm2 = _run_partition(src_pts, tgt_pts, src_node_xyz, tgt_node_xyz)
    src_nm, src_knn_idx, src_knn_m, src_knn_pts = _finish_partition(
        knn2[0], knnx2[0], nm2[0])
    tgt_nm, tgt_knn_idx, tgt_knn_m, tgt_knn_pts = _finish_partition(
        knn2[1], knnx2[1], nm2[1])

    tgt_corr_idx, src_corr_idx = _run_coarse(tgt_node_f, src_node_f, tgt_nm, src_nm)

    src_ck_idx = src_knn_idx[src_corr_idx]
    tgt_ck_idx = tgt_knn_idx[tgt_corr_idx]
    src_ck_m = src_knn_m[src_corr_idx]
    tgt_ck_m = tgt_knn_m[tgt_corr_idx]
    src_ck_pts = src_knn_pts[src_corr_idx]
    tgt_ck_pts = tgt_knn_pts[tgt_corr_idx]

    src_ck_f = jnp.where(src_ck_m[:, :, None],
                         (src_ck_pts @ W_point) @ W_fine + b_fine, 0.0)
    tgt_ck_f = jnp.where(tgt_ck_m[:, :, None],
                         (tgt_ck_pts @ W_point) @ W_fine + b_fine, 0.0)

    ms = jnp.einsum('bnd,bmd->bnm', tgt_ck_f, src_ck_f) / (W_fine.shape[1] ** 0.5)
    ms, corr_scores, tgt_corr_pts, src_corr_pts = _run_ot(
        ms, tgt_ck_m, src_ck_m, alpha, tgt_ck_pts, src_ck_pts)
    return ms, tgt_corr_pts, src_corr_pts, corr_scores


# speedup vs baseline: 1.1363x; 1.1363x over previous
import functools

import jax
import jax.numpy as jnp
from jax.experimental import pallas as pl
from jax.experimental.pallas import tpu as pltpu

N_POINTS = 20000
NUM_NODES = 512
POINT_LIMIT = 64
NUM_CORR = 256
FINE_TOPK = 3
OT_ITERS = 100

_BB = 32


def _lse(x, axis):
    m = jnp.max(x, axis=axis)
    return jnp.log(jnp.sum(jnp.exp(x - jnp.expand_dims(m, axis)), axis=axis)) + m


def _coarse_kernel(tf_ref, sf_ref, tm_ref, sm_ref, fidx_ref):
    Mn = tf_ref.shape[0]
    s1 = jax.lax.dot_general(tf_ref[...], sf_ref[...], (((1,), (1,)), ((), ())),
                             preferred_element_type=jnp.float32)
    d = jnp.maximum(2.0 - 2.0 * s1, 0.0)
    s = jnp.exp(-d)
    s = (s / jnp.sum(s, axis=1, keepdims=True)) * (s / jnp.sum(s, axis=0, keepdims=True))
    valid = (tm_ref[...] > 0.0) & (jnp.transpose(sm_ref[...]) > 0.0)
    s = jnp.where(valid, s, 0.0)
    flat = (jax.lax.broadcasted_iota(jnp.int32, (Mn, Mn), 0) * Mn
            + jax.lax.broadcasted_iota(jnp.int32, (Mn, Mn), 1)).astype(jnp.float32)
    kcol = jax.lax.broadcasted_iota(jnp.int32, (1, NUM_CORR), 1)

    def body(k, carry):
        s, acc = carry
        m1 = jnp.max(jnp.max(s, axis=1, keepdims=True), axis=0, keepdims=True)
        eq = s == m1
        fmin = jnp.min(jnp.min(jnp.where(eq, flat, 3.0e8), axis=1, keepdims=True),
                       axis=0, keepdims=True)
        s = jnp.where(eq & (flat == fmin), -1.0, s)
        return s, jnp.where(kcol == k, fmin, acc)

    _, acc = jax.lax.fori_loop(0, NUM_CORR, body,
                               (s, jnp.zeros((1, NUM_CORR), jnp.float32)))
    fidx_ref[...] = acc


def _run_coarse(tgt_node_f, src_node_f, tgt_nm, src_nm):
    M = NUM_NODES
    fidx = pl.pallas_call(
        _coarse_kernel,
        out_shape=jax.ShapeDtypeStruct((1, NUM_CORR), jnp.float32),
    )(tgt_node_f, src_node_f,
      tgt_nm.astype(jnp.float32).reshape(M, 1),
      src_nm.astype(jnp.float32).reshape(M, 1))
    flat_idx = fidx[0].astype(jnp.int32)
    return flat_idx // M, flat_idx % M


def _ot_kernel(scores_ref, rowm_ref, colm_ref, alpha_ref, tpts_ref, spts_ref,
               out_ref, cs_ref, tcp_ref, scp_ref):
    scores = scores_ref[...]
    rowm = rowm_ref[...]
    colm = colm_ref[...]
    alpha = alpha_ref[0, 0]
    b = scores.shape[0]
    m1 = scores.shape[1]

    ridx = jax.lax.broadcasted_iota(jnp.int32, (b, m1, m1), 1)
    cidx = jax.lax.broadcasted_iota(jnp.int32, (b, m1, m1), 2)
    is_pad = (ridx == m1 - 1) | (cidx == m1 - 1)
    padded = jnp.where(is_pad, alpha, scores)

    ridx2 = jax.lax.broadcasted_iota(jnp.int32, (b, m1), 1)
    prm = jnp.where(ridx2 == m1 - 1, 1.0, jnp.pad(rowm, ((0, 0), (0, 1))))
    pcm = jnp.where(ridx2 == m1 - 1, 1.0, jnp.pad(colm, ((0, 0), (0, 1))))
    sm = (prm[:, :, None] * pcm[:, None, :]) > 0.5
    padded = jnp.where(sm, padded, -1e5)

    nvr = jnp.maximum(jnp.sum(rowm, axis=1), 1.0)
    nvc = jnp.maximum(jnp.sum(colm, axis=1), 1.0)
    norm = -jnp.log(nvr + nvc)
    log_mu = jnp.where(ridx2 == m1 - 1, (jnp.log(nvc) + norm)[:, None], norm[:, None])
    log_nu = jnp.where(ridx2 == m1 - 1, (jnp.log(nvr) + norm)[:, None], norm[:, None])

    def step(i, uv):
        u, v = uv
        u = log_mu - _lse(padded + v[:, None, :], axis=2)
        v = log_nu - _lse(padded + u[:, :, None], axis=1)
        return (u, v)

    u0 = jnp.zeros((b, m1), jnp.float32)
    v0 = jnp.zeros((b, m1), jnp.float32)
    u, v = jax.lax.fori_loop(0, OT_ITERS, step, (u0, v0))
    out = padded + u[:, :, None] + v[:, None, :] - norm[:, None, None]
    out_ref[...] = out

    P = POINT_LIMIT
    fm = jnp.exp(out[:, :P, :P])
    fmask = (rowm[:, :, None] > 0.5) & (colm[:, None, :] > 0.5)
    fm = jnp.where(fmask, fm, 0.0)
    flat = (jax.lax.broadcasted_iota(jnp.int32, (b, P, P), 1) * P
            + jax.lax.broadcasted_iota(jnp.int32, (b, P, P), 2)).astype(jnp.float32)
    kio = jax.lax.broadcasted_iota(jnp.int32, (b, 1, P), 2).astype(jnp.float32)
    tio = jax.lax.broadcasted_iota(jnp.int32, (b, FINE_TOPK), 1)
    tio3 = jax.lax.broadcasted_iota(jnp.int32, (b, FINE_TOPK, 3), 1)
    cs = jnp.zeros((b, FINE_TOPK), jnp.float32)
    tcp = jnp.zeros((b, FINE_TOPK, 3), jnp.float32)
    scp = jnp.zeros((b, FINE_TOPK, 3), jnp.float32)
    for t in range(FINE_TOPK):
        m2 = jnp.max(jnp.max(fm, axis=2, keepdims=True), axis=1, keepdims=True)
        eq = fm == m2
        fmin = jnp.min(jnp.min(jnp.where(eq, flat, 3.0e8), axis=2, keepdims=True),
                       axis=1, keepdims=True)
        fm = jnp.where(eq & (flat == fmin), -1.0, fm)
        ri = jnp.floor(fmin * (1.0 / P))
        ci = fmin - ri * P
        tsel = jnp.sum(jnp.where(kio == ri[:, :, 0][:, :, None], tpts_ref[...], 0.0),
                       axis=2)
        ssel = jnp.sum(jnp.where(kio == ci[:, :, 0][:, :, None], spts_ref[...], 0.0),
                       axis=2)
        cs = jnp.where(tio == t, m2[:, :, 0], cs)
        tcp = jnp.where(tio3 == t, tsel[:, None, :], tcp)
        scp = jnp.where(tio3 == t, ssel[:, None, :], scp)
    cs_ref[...] = cs
    tcp_ref[...] = tcp
    scp_ref[...] = scp


def _run_ot(scores, row_masks, col_masks, alpha, tgt_ck_pts, src_ck_pts):
    B = scores.shape[0]
    scores65 = jnp.pad(scores, ((0, 0), (0, 1), (0, 1)))
    rowm = row_masks.astype(jnp.float32)
    colm = col_masks.astype(jnp.float32)
    alpha_arr = jnp.full((1, 1), alpha, jnp.float32)
    tpts = jnp.swapaxes(tgt_ck_pts, 1, 2)
    spts = jnp.swapaxes(src_ck_pts, 1, 2)
    grid = (B // _BB,)
    return pl.pallas_call(
        _ot_kernel,
        grid=grid,
        in_specs=[
            pl.BlockSpec((_BB, 65, 65), lambda i: (i, 0, 0)),
            pl.BlockSpec((_BB, 64), lambda i: (i, 0)),
            pl.BlockSpec((_BB, 64), lambda i: (i, 0)),
            pl.BlockSpec(memory_space=pltpu.SMEM),
            pl.BlockSpec((_BB, 3, 64), lambda i: (i, 0, 0)),
            pl.BlockSpec((_BB, 3, 64), lambda i: (i, 0, 0)),
        ],
        out_specs=[
            pl.BlockSpec((_BB, 65, 65), lambda i: (i, 0, 0)),
            pl.BlockSpec((_BB, FINE_TOPK), lambda i: (i, 0)),
            pl.BlockSpec((_BB, FINE_TOPK, 3), lambda i: (i, 0, 0)),
            pl.BlockSpec((_BB, FINE_TOPK, 3), lambda i: (i, 0, 0)),
        ],
        out_shape=[
            jax.ShapeDtypeStruct((B, 65, 65), jnp.float32),
            jax.ShapeDtypeStruct((B, FINE_TOPK), jnp.float32),
            jax.ShapeDtypeStruct((B, FINE_TOPK, 3), jnp.float32),
            jax.ShapeDtypeStruct((B, FINE_TOPK, 3), jnp.float32),
        ],
        compiler_params=pltpu.CompilerParams(
            dimension_semantics=("parallel",),
        ),
    )(scores65, rowm, colm, alpha_arr, tpts, spts)


_NPAD = 20480
_PCH = 2048
_INF = 3.0e38


def _partition_kernel(pts_ref, nodes_ref, knn_ref, knnx_ref, cnt_ref, key_ref,
                      parg_ref):
    M = nodes_ref.shape[1]
    nx = nodes_ref[0, :, 0:1]
    ny = nodes_ref[0, :, 1:2]
    nz = nodes_ref[0, :, 2:3]
    cnt_ref[0, :, :] = jnp.zeros((M, 1), jnp.float32)

    def chunk_body(c, _):
        base = c * _PCH
        px = pts_ref[0, 0:1, pl.ds(base, _PCH)]
        py = pts_ref[0, 1:2, pl.ds(base, _PCH)]
        pz = pts_ref[0, 2:3, pl.ds(base, _PCH)]
        dx = nx - px
        dy = ny - py
        dz = nz - pz
        sq = (dx * dx + dy * dy) + dz * dz
        jglob = (jax.lax.broadcasted_iota(jnp.int32, (M, _PCH), 1)
                 + base).astype(jnp.float32)
        valid = jglob < float(N_POINTS)
        sq = jnp.where(valid, sq, _INF)
        key_ref[:, pl.ds(base, _PCH)] = sq
        colmin = jnp.min(sq, axis=0, keepdims=True)
        m_iota = jax.lax.broadcasted_iota(jnp.int32, (M, _PCH), 0).astype(jnp.float32)
        colarg = jnp.min(jnp.where(sq == colmin, m_iota, 1e9), axis=0,
                         keepdims=True)
        contrib = jnp.where((m_iota == colarg) & valid, 1.0, 0.0)
        cnt_ref[0, :, :] = cnt_ref[0, :, :] + jnp.sum(contrib, axis=1, keepdims=True)
        parg_ref[0:1, pl.ds(base, _PCH)] = colarg
        return 0

    jax.lax.fori_loop(0, _NPAD // _PCH, chunk_body, 0)

    kcol = jax.lax.broadcasted_iota(jnp.int32, (M, POINT_LIMIT), 1)
    nch = _NPAD // _PCH
    jio0 = jax.lax.broadcasted_iota(jnp.int32, (M, _PCH), 1).astype(jnp.float32)

    _ECH = 4096
    ech = _NPAD // _ECH
    jio1 = jax.lax.broadcasted_iota(jnp.int32, (M, _ECH), 1).astype(jnp.float32)

    def ext_body(k, acc):
        def min_c(c, m1):
            keys = key_ref[:, pl.ds(c * _ECH, _ECH)]
            return jnp.minimum(m1, jnp.min(keys, axis=1, keepdims=True))

        m1 = jax.lax.fori_loop(0, ech, min_c, jnp.full((M, 1), _INF, jnp.float32))

        def jmin_c(c, jm):
            keys = key_ref[:, pl.ds(c * _ECH, _ECH)]
            jio = jio1 + (c * _ECH).astype(jnp.float32)
            cand = jnp.min(jnp.where(keys == m1, jio, _INF), axis=1, keepdims=True)
            return jnp.minimum(jm, cand)

        jmin = jax.lax.fori_loop(0, ech, jmin_c, jnp.full((M, 1), _INF, jnp.float32))

        def upd_c(c, _):
            keys = key_ref[:, pl.ds(c * _ECH, _ECH)]
            jio = jio1 + (c * _ECH).astype(jnp.float32)
            key_ref[:, pl.ds(c * _ECH, _ECH)] = jnp.where(
                (keys == m1) & (jio == jmin), _INF, keys)
            return 0

        jax.lax.fori_loop(0, ech, upd_c, 0)
        return jnp.where(kcol == k, jmin, acc)

    acc = jax.lax.fori_loop(
        0, POINT_LIMIT, ext_body, jnp.zeros((M, POINT_LIMIT), jnp.float32))
    knn_ref[0, :, :] = acc

    G = _NPAD // 128

    def table_gl(row):
        return row.reshape(G, 128)

    tbl = jnp.concatenate(
        [table_gl(pts_ref[0, 0:1, :]), table_gl(pts_ref[0, 1:2, :]),
         table_gl(pts_ref[0, 2:3, :]), table_gl(parg_ref[0:1, :])], axis=0)
    tbl_hi = tbl.astype(jnp.bfloat16)
    tbl_mid = (tbl - tbl_hi.astype(jnp.float32)).astype(jnp.bfloat16)
    tbl_lo = (tbl - tbl_hi.astype(jnp.float32)
              - tbl_mid.astype(jnp.float32)).astype(jnp.bfloat16)
    acc_t = jnp.transpose(acc)
    dn = (((1,), (0,)), ((), ()))
    lio = jax.lax.broadcasted_iota(jnp.int32, (128, M), 0).astype(jnp.float32)
    gio = jax.lax.broadcasted_iota(jnp.int32, (G, M), 0).astype(jnp.float32)
    for k in range(POINT_LIMIT):
        row = jax.lax.slice(acc_t, (k, 0), (k + 1, M))
        hi = jnp.floor(row * (1.0 / 128.0))
        lo = row - hi * 128.0
        oh_lo = (lio == lo).astype(jnp.bfloat16)
        tmp = (jax.lax.dot_general(tbl_hi, oh_lo, dn,
                                   preferred_element_type=jnp.float32)
               + jax.lax.dot_general(tbl_mid, oh_lo, dn,
                                     preferred_element_type=jnp.float32)
               + jax.lax.dot_general(tbl_lo, oh_lo, dn,
                                     preferred_element_type=jnp.float32))
        oh_hi = jnp.where(gio == hi, 1.0, 0.0)
        for col in range(4):
            piece = jax.lax.slice(tmp, (col * G, 0), ((col + 1) * G, M))
            val = jnp.sum(piece * oh_hi, axis=0, keepdims=True)
            knnx_ref[0, col, k:k + 1, :] = val


def _run_partition(src_pts, tgt_pts, nodes_src, nodes_tgt):
    def prep(p):
        return jnp.pad(p, ((0, _NPAD - N_POINTS), (0, 0))).T
    pts2 = jnp.stack([prep(src_pts), prep(tgt_pts)])
    nodes2 = jnp.stack([nodes_src, nodes_tgt])
    M = NUM_NODES
    knn_f, knnx, cnt = pl.pallas_call(
        _partition_kernel,
        grid=(2,),
        in_specs=[
            pl.BlockSpec((1, 3, _NPAD), lambda i: (i, 0, 0)),
            pl.BlockSpec((1, M, 3), lambda i: (i, 0, 0)),
        ],
        out_specs=[
            pl.BlockSpec((1, M, POINT_LIMIT), lambda i: (i, 0, 0)),
            pl.BlockSpec((1, 4, POINT_LIMIT, M), lambda i: (i, 0, 0, 0)),
            pl.BlockSpec((1, M, 1), lambda i: (i, 0, 0)),
        ],
        out_shape=[
            jax.ShapeDtypeStruct((2, M, POINT_LIMIT), jnp.float32),
            jax.ShapeDtypeStruct((2, 4, POINT_LIMIT, M), jnp.float32),
            jax.ShapeDtypeStruct((2, M, 1), jnp.float32),
        ],
        scratch_shapes=[pltpu.VMEM((M, _NPAD), jnp.float32),
                        pltpu.VMEM((1, _NPAD), jnp.float32)],
        compiler_params=pltpu.CompilerParams(
            dimension_semantics=("parallel",),
        ),
    )(pts2, nodes2)
    knn_idx = knn_f.astype(jnp.int32)
    node_masks = cnt[:, :, 0] > 0
    return knn_idx, jnp.swapaxes(knnx, 2, 3), node_masks


def _finish_partition(knn_idx, knnx, node_masks):
    knn_node = knnx[3].astype(jnp.int32)
    knn_masks = (knn_node == jnp.arange(NUM_NODES)[:, None]) & node_masks[:, None]
    knn_idx = jnp.where(knn_masks, knn_idx, N_POINTS)
    knn_pts = jnp.where(knn_masks[:, :, None],
                        jnp.stack([knnx[0], knnx[1], knnx[2]], axis=-1), 0.0)
    return node_masks, knn_idx, knn_masks, knn_pts


def kernel(src_pcd, tgt_pcd, src_feats, tgt_feats, src_normals, tgt_normals,
           rot, trans, src_raw_pcd, W_node, W_point, W_coarse, b_coarse,
           W_fine, b_fine, alpha):
    src_pts, tgt_pts = src_pcd, tgt_pcd
    stride = src_pts.shape[0] // NUM_NODES
    node_idx = jnp.arange(NUM_NODES) * stride
    src_node_xyz = src_pts[node_idx]
    tgt_node_xyz = tgt_pts[node_idx]
    src_node_f = src_node_xyz @ W_node
    tgt_node_f = tgt_node_xyz @ W_node

    def l2n(x):
        return x / (jnp.linalg.norm(x, axis=1, keepdims=True) + 1e-12)

    src_node_f = l2n(src_node_f @ W_coarse + b_coarse)
    tgt_node_f = l2n(tgt_node_f @ W_coarse + b_coarse)

    knn2, knnx2, nm2 = _run_partition(src_pts, tgt_pts, src_node_xyz, tgt_node_xyz)
    src_nm, src_knn_idx, src_knn_m, src_knn_pts = _finish_partition(
        knn2[0], knnx2[0], nm2[0])
    tgt_nm, tgt_knn_idx, tgt_knn_m, tgt_knn_pts = _finish_partition(
        knn2[1], knnx2[1], nm2[1])

    tgt_corr_idx, src_corr_idx = _run_coarse(tgt_node_f, src_node_f, tgt_nm, src_nm)

    src_ck_idx = src_knn_idx[src_corr_idx]
    tgt_ck_idx = tgt_knn_idx[tgt_corr_idx]
    src_ck_m = src_knn_m[src_corr_idx]
    tgt_ck_m = tgt_knn_m[tgt_corr_idx]
    src_ck_pts = src_knn_pts[src_corr_idx]
    tgt_ck_pts = tgt_knn_pts[tgt_corr_idx]

    src_ck_f = jnp.where(src_ck_m[:, :, None],
                         (src_ck_pts @ W_point) @ W_fine + b_fine, 0.0)
    tgt_ck_f = jnp.where(tgt_ck_m[:, :, None],
                         (tgt_ck_pts @ W_point) @ W_fine + b_fine, 0.0)

    ms = jnp.einsum('bnd,bmd->bnm', tgt_ck_f, src_ck_f) / (W_fine.shape[1] ** 0.5)
    ms, corr_scores, tgt_corr_pts, src_corr_pts = _run_ot(
        ms, tgt_ck_m, src_ck_m, alpha, tgt_ck_pts, src_ck_pts)
    return ms, tgt_corr_pts, src_corr_pts, corr_scores


# speedup vs baseline: 1.1364x; 1.0001x over previous
import jax
import jax.numpy as jnp
from jax.experimental import pallas as pl
from jax.experimental.pallas import tpu as pltpu

N_POINTS = 20000
NUM_NODES = 512
POINT_LIMIT = 64
NUM_CORR = 256
FINE_TOPK = 3
OT_ITERS = 100

_BB = 32


def _lse(x, axis):
    m = jnp.max(x, axis=axis)
    return jnp.log(jnp.sum(jnp.exp(x - jnp.expand_dims(m, axis)), axis=axis)) + m


def _coarse_kernel(tf_ref, sf_ref, tm_ref, sm_ref, fidx_ref):
    Mn = tf_ref.shape[0]
    s1 = jax.lax.dot_general(tf_ref[...], sf_ref[...], (((1,), (1,)), ((), ())),
                             preferred_element_type=jnp.float32)
    d = jnp.maximum(2.0 - 2.0 * s1, 0.0)
    s = jnp.exp(-d)
    s = (s / jnp.sum(s, axis=1, keepdims=True)) * (s / jnp.sum(s, axis=0, keepdims=True))
    valid = (tm_ref[...] > 0.0) & (jnp.transpose(sm_ref[...]) > 0.0)
    s = jnp.where(valid, s, 0.0)
    flat = (jax.lax.broadcasted_iota(jnp.int32, (Mn, Mn), 0) * Mn
            + jax.lax.broadcasted_iota(jnp.int32, (Mn, Mn), 1)).astype(jnp.float32)
    kcol = jax.lax.broadcasted_iota(jnp.int32, (1, NUM_CORR), 1)

    def body(k, carry):
        s, acc = carry
        m1 = jnp.max(jnp.max(s, axis=1, keepdims=True), axis=0, keepdims=True)
        eq = s == m1
        fmin = jnp.min(jnp.min(jnp.where(eq, flat, 3.0e8), axis=1, keepdims=True),
                       axis=0, keepdims=True)
        s = jnp.where(eq & (flat == fmin), -1.0, s)
        return s, jnp.where(kcol == k, fmin, acc)

    _, acc = jax.lax.fori_loop(0, NUM_CORR, body,
                               (s, jnp.zeros((1, NUM_CORR), jnp.float32)))
    fidx_ref[...] = acc


def _run_coarse(tgt_node_f, src_node_f, tgt_nm, src_nm):
    M = NUM_NODES
    fidx = pl.pallas_call(
        _coarse_kernel,
        out_shape=jax.ShapeDtypeStruct((1, NUM_CORR), jnp.float32),
    )(tgt_node_f, src_node_f,
      tgt_nm.astype(jnp.float32).reshape(M, 1),
      src_nm.astype(jnp.float32).reshape(M, 1))
    flat_idx = fidx[0].astype(jnp.int32)
    return flat_idx // M, flat_idx % M


def _ot_kernel(scores_ref, rowm_ref, colm_ref, alpha_ref, tpts_ref, spts_ref,
               out_ref, cs_ref, tcp_ref, scp_ref):
    scores = scores_ref[...]
    rowm = rowm_ref[...]
    colm = colm_ref[...]
    alpha = alpha_ref[0, 0]
    b = scores.shape[0]
    m1 = scores.shape[1]

    ridx = jax.lax.broadcasted_iota(jnp.int32, (b, m1, m1), 1)
    cidx = jax.lax.broadcasted_iota(jnp.int32, (b, m1, m1), 2)
    is_pad = (ridx == m1 - 1) | (cidx == m1 - 1)
    padded = jnp.where(is_pad, alpha, scores)

    ridx2 = jax.lax.broadcasted_iota(jnp.int32, (b, m1), 1)
    prm = jnp.where(ridx2 == m1 - 1, 1.0, jnp.pad(rowm, ((0, 0), (0, 1))))
    pcm = jnp.where(ridx2 == m1 - 1, 1.0, jnp.pad(colm, ((0, 0), (0, 1))))
    sm = (prm[:, :, None] * pcm[:, None, :]) > 0.5
    padded = jnp.where(sm, padded, -1e5)

    nvr = jnp.maximum(jnp.sum(rowm, axis=1), 1.0)
    nvc = jnp.maximum(jnp.sum(colm, axis=1), 1.0)
    norm = -jnp.log(nvr + nvc)
    log_mu = jnp.where(ridx2 == m1 - 1, (jnp.log(nvc) + norm)[:, None], norm[:, None])
    log_nu = jnp.where(ridx2 == m1 - 1, (jnp.log(nvr) + norm)[:, None], norm[:, None])

    def step(i, uv):
        u, v = uv
        u = log_mu - _lse(padded + v[:, None, :], axis=2)
        v = log_nu - _lse(padded + u[:, :, None], axis=1)
        return (u, v)

    u0 = jnp.zeros((b, m1), jnp.float32)
    v0 = jnp.zeros((b, m1), jnp.float32)
    u, v = jax.lax.fori_loop(0, OT_ITERS, step, (u0, v0))
    out = padded + u[:, :, None] + v[:, None, :] - norm[:, None, None]
    out_ref[...] = out

    P = POINT_LIMIT
    fm = jnp.exp(out[:, :P, :P])
    fmask = (rowm[:, :, None] > 0.5) & (colm[:, None, :] > 0.5)
    fm = jnp.where(fmask, fm, 0.0)
    flat = (jax.lax.broadcasted_iota(jnp.int32, (b, P, P), 1) * P
            + jax.lax.broadcasted_iota(jnp.int32, (b, P, P), 2)).astype(jnp.float32)
    kio = jax.lax.broadcasted_iota(jnp.int32, (b, 1, P), 2).astype(jnp.float32)
    tio = jax.lax.broadcasted_iota(jnp.int32, (b, FINE_TOPK), 1)
    tio3 = jax.lax.broadcasted_iota(jnp.int32, (b, FINE_TOPK, 3), 1)
    cs = jnp.zeros((b, FINE_TOPK), jnp.float32)
    tcp = jnp.zeros((b, FINE_TOPK, 3), jnp.float32)
    scp = jnp.zeros((b, FINE_TOPK, 3), jnp.float32)
    for t in range(FINE_TOPK):
        m2 = jnp.max(jnp.max(fm, axis=2, keepdims=True), axis=1, keepdims=True)
        eq = fm == m2
        fmin = jnp.min(jnp.min(jnp.where(eq, flat, 3.0e8), axis=2, keepdims=True),
                       axis=1, keepdims=True)
        fm = jnp.where(eq & (flat == fmin), -1.0, fm)
        ri = jnp.floor(fmin * (1.0 / P))
        ci = fmin - ri * P
        tsel = jnp.sum(jnp.where(kio == ri[:, :, 0][:, :, None], tpts_ref[...], 0.0),
                       axis=2)
        ssel = jnp.sum(jnp.where(kio == ci[:, :, 0][:, :, None], spts_ref[...], 0.0),
                       axis=2)
        cs = jnp.where(tio == t, m2[:, :, 0], cs)
        tcp = jnp.where(tio3 == t, tsel[:, None, :], tcp)
        scp = jnp.where(tio3 == t, ssel[:, None, :], scp)
    cs_ref[...] = cs
    tcp_ref[...] = tcp
    scp_ref[...] = scp


def _run_ot(scores, row_masks, col_masks, alpha, tgt_ck_pts, src_ck_pts):
    B = scores.shape[0]
    scores65 = jnp.pad(scores, ((0, 0), (0, 1), (0, 1)))
    rowm = row_masks.astype(jnp.float32)
    colm = col_masks.astype(jnp.float32)
    alpha_arr = jnp.full((1, 1), alpha, jnp.float32)
    tpts = jnp.swapaxes(tgt_ck_pts, 1, 2)
    spts = jnp.swapaxes(src_ck_pts, 1, 2)
    grid = (B // _BB,)
    return pl.pallas_call(
        _ot_kernel,
        grid=grid,
        in_specs=[
            pl.BlockSpec((_BB, 65, 65), lambda i: (i, 0, 0)),
            pl.BlockSpec((_BB, 64), lambda i: (i, 0)),
            pl.BlockSpec((_BB, 64), lambda i: (i, 0)),
            pl.BlockSpec(memory_space=pltpu.SMEM),
            pl.BlockSpec((_BB, 3, 64), lambda i: (i, 0, 0)),
            pl.BlockSpec((_BB, 3, 64), lambda i: (i, 0, 0)),
        ],
        out_specs=[
            pl.BlockSpec((_BB, 65, 65), lambda i: (i, 0, 0)),
            pl.BlockSpec((_BB, FINE_TOPK), lambda i: (i, 0)),
            pl.BlockSpec((_BB, FINE_TOPK, 3), lambda i: (i, 0, 0)),
            pl.BlockSpec((_BB, FINE_TOPK, 3), lambda i: (i, 0, 0)),
        ],
        out_shape=[
            jax.ShapeDtypeStruct((B, 65, 65), jnp.float32),
            jax.ShapeDtypeStruct((B, FINE_TOPK), jnp.float32),
            jax.ShapeDtypeStruct((B, FINE_TOPK, 3), jnp.float32),
            jax.ShapeDtypeStruct((B, FINE_TOPK, 3), jnp.float32),
        ],
        compiler_params=pltpu.CompilerParams(
            dimension_semantics=("parallel",),
        ),
    )(scores65, rowm, colm, alpha_arr, tpts, spts)


_NPAD = 20480
_PCH = 2048
_INF = 3.0e38


def _partition_kernel(pts_ref, nodes_ref, knn_ref, knnx_ref, cnt_ref, key_ref,
                      parg_ref):
    M = nodes_ref.shape[1]
    nx = nodes_ref[0, :, 0:1]
    ny = nodes_ref[0, :, 1:2]
    nz = nodes_ref[0, :, 2:3]
    cnt_ref[0, :, :] = jnp.zeros((M, 1), jnp.float32)

    def chunk_body(c, _):
        base = c * _PCH
        px = pts_ref[0, 0:1, pl.ds(base, _PCH)]
        py = pts_ref[0, 1:2, pl.ds(base, _PCH)]
        pz = pts_ref[0, 2:3, pl.ds(base, _PCH)]
        dx = nx - px
        dy = ny - py
        dz = nz - pz
        sq = (dx * dx + dy * dy) + dz * dz
        jglob = (jax.lax.broadcasted_iota(jnp.int32, (M, _PCH), 1)
                 + base).astype(jnp.float32)
        valid = jglob < float(N_POINTS)
        sq = jnp.where(valid, sq, _INF)
        key_ref[:, pl.ds(base, _PCH)] = sq
        colmin = jnp.min(sq, axis=0, keepdims=True)
        m_iota = jax.lax.broadcasted_iota(jnp.int32, (M, _PCH), 0).astype(jnp.float32)
        colarg = jnp.min(jnp.where(sq == colmin, m_iota, 1e9), axis=0,
                         keepdims=True)
        contrib = jnp.where((m_iota == colarg) & valid, 1.0, 0.0)
        cnt_ref[0, :, :] = cnt_ref[0, :, :] + jnp.sum(contrib, axis=1, keepdims=True)
        parg_ref[0:1, pl.ds(base, _PCH)] = colarg
        return 0

    jax.lax.fori_loop(0, _NPAD // _PCH, chunk_body, 0)

    kcol = jax.lax.broadcasted_iota(jnp.int32, (M, POINT_LIMIT), 1)
    _ECH = 4096
    ech = _NPAD // _ECH
    jio1 = jax.lax.broadcasted_iota(jnp.int32, (M, _ECH), 1).astype(jnp.float32)

    def ext_body(k, acc):
        def min_c(c, m1):
            keys = key_ref[:, pl.ds(c * _ECH, _ECH)]
            return jnp.minimum(m1, jnp.min(keys, axis=1, keepdims=True))

        m1 = jax.lax.fori_loop(0, ech, min_c, jnp.full((M, 1), _INF, jnp.float32))

        def jmin_c(c, jm):
            keys = key_ref[:, pl.ds(c * _ECH, _ECH)]
            jio = jio1 + (c * _ECH).astype(jnp.float32)
            cand = jnp.min(jnp.where(keys == m1, jio, _INF), axis=1, keepdims=True)
            return jnp.minimum(jm, cand)

        jmin = jax.lax.fori_loop(0, ech, jmin_c, jnp.full((M, 1), _INF, jnp.float32))

        def upd_c(c, _):
            keys = key_ref[:, pl.ds(c * _ECH, _ECH)]
            jio = jio1 + (c * _ECH).astype(jnp.float32)
            key_ref[:, pl.ds(c * _ECH, _ECH)] = jnp.where(
                (keys == m1) & (jio == jmin), _INF, keys)
            return 0

        jax.lax.fori_loop(0, ech, upd_c, 0)
        return jnp.where(kcol == k, jmin, acc)

    acc = jax.lax.fori_loop(
        0, POINT_LIMIT, ext_body, jnp.zeros((M, POINT_LIMIT), jnp.float32))
    knn_ref[0, :, :] = acc

    G = _NPAD // 128

    def table_gl(row):
        return row.reshape(G, 128)

    tbl = jnp.concatenate(
        [table_gl(pts_ref[0, 0:1, :]), table_gl(pts_ref[0, 1:2, :]),
         table_gl(pts_ref[0, 2:3, :]), table_gl(parg_ref[0:1, :])], axis=0)
    tbl_hi = tbl.astype(jnp.bfloat16)
    tbl_mid = (tbl - tbl_hi.astype(jnp.float32)).astype(jnp.bfloat16)
    tbl_lo = (tbl - tbl_hi.astype(jnp.float32)
              - tbl_mid.astype(jnp.float32)).astype(jnp.bfloat16)
    acc_t = jnp.transpose(acc)
    dn = (((1,), (0,)), ((), ()))
    lio = jax.lax.broadcasted_iota(jnp.int32, (128, M), 0).astype(jnp.float32)
    gio = jax.lax.broadcasted_iota(jnp.int32, (G, M), 0).astype(jnp.float32)
    for k in range(POINT_LIMIT):
        row = jax.lax.slice(acc_t, (k, 0), (k + 1, M))
        hi = jnp.floor(row * (1.0 / 128.0))
        lo = row - hi * 128.0
        oh_lo = (lio == lo).astype(jnp.bfloat16)
        tmp = (jax.lax.dot_general(tbl_hi, oh_lo, dn,
                                   preferred_element_type=jnp.float32)
               + jax.lax.dot_general(tbl_mid, oh_lo, dn,
                                     preferred_element_type=jnp.float32)
               + jax.lax.dot_general(tbl_lo, oh_lo, dn,
                                     preferred_element_type=jnp.float32))
        oh_hi = jnp.where(gio == hi, 1.0, 0.0)
        for col in range(4):
            piece = jax.lax.slice(tmp, (col * G, 0), ((col + 1) * G, M))
            val = jnp.sum(piece * oh_hi, axis=0, keepdims=True)
            knnx_ref[0, col, k:k + 1, :] = val


def _run_partition(src_pts, tgt_pts, nodes_src, nodes_tgt):
    def prep(p):
        return jnp.pad(p, ((0, _NPAD - N_POINTS), (0, 0))).T
    pts2 = jnp.stack([prep(src_pts), prep(tgt_pts)])
    nodes2 = jnp.stack([nodes_src, nodes_tgt])
    M = NUM_NODES
    knn_f, knnx, cnt = pl.pallas_call(
        _partition_kernel,
        grid=(2,),
        in_specs=[
            pl.BlockSpec((1, 3, _NPAD), lambda i: (i, 0, 0)),
            pl.BlockSpec((1, M, 3), lambda i: (i, 0, 0)),
        ],
        out_specs=[
            pl.BlockSpec((1, M, POINT_LIMIT), lambda i: (i, 0, 0)),
            pl.BlockSpec((1, 4, POINT_LIMIT, M), lambda i: (i, 0, 0, 0)),
            pl.BlockSpec((1, M, 1), lambda i: (i, 0, 0)),
        ],
        out_shape=[
            jax.ShapeDtypeStruct((2, M, POINT_LIMIT), jnp.float32),
            jax.ShapeDtypeStruct((2, 4, POINT_LIMIT, M), jnp.float32),
            jax.ShapeDtypeStruct((2, M, 1), jnp.float32),
        ],
        scratch_shapes=[pltpu.VMEM((M, _NPAD), jnp.float32),
                        pltpu.VMEM((1, _NPAD), jnp.float32)],
        compiler_params=pltpu.CompilerParams(
            dimension_semantics=("parallel",),
        ),
    )(pts2, nodes2)
    knn_idx = knn_f.astype(jnp.int32)
    node_masks = cnt[:, :, 0] > 0
    return knn_idx, jnp.swapaxes(knnx, 2, 3), node_masks


def _finish_partition(knn_idx, knnx, node_masks):
    knn_node = knnx[3].astype(jnp.int32)
    knn_masks = (knn_node == jnp.arange(NUM_NODES)[:, None]) & node_masks[:, None]
    knn_idx = jnp.where(knn_masks, knn_idx, N_POINTS)
    knn_pts = jnp.where(knn_masks[:, :, None],
                        jnp.stack([knnx[0], knnx[1], knnx[2]], axis=-1), 0.0)
    return node_masks, knn_idx, knn_masks, knn_pts


def kernel(src_pcd, tgt_pcd, src_feats, tgt_feats, src_normals, tgt_normals,
           rot, trans, src_raw_pcd, W_node, W_point, W_coarse, b_coarse,
           W_fine, b_fine, alpha):
    src_pts, tgt_pts = src_pcd, tgt_pcd
    stride = src_pts.shape[0] // NUM_NODES
    node_idx = jnp.arange(NUM_NODES) * stride
    src_node_xyz = src_pts[node_idx]
    tgt_node_xyz = tgt_pts[node_idx]
    src_node_f = src_node_xyz @ W_node
    tgt_node_f = tgt_node_xyz @ W_node

    def l2n(x):
        return x / (jnp.linalg.norm(x, axis=1, keepdims=True) + 1e-12)

    src_node_f = l2n(src_node_f @ W_coarse + b_coarse)
    tgt_node_f = l2n(tgt_node_f @ W_coarse + b_coarse)

    knn2, knnx2, nm2 = _run_partition(src_pts, tgt_pts, src_node_xyz, tgt_node_xyz)
    src_nm, src_knn_idx, src_knn_m, src_knn_pts = _finish_partition(
        knn2[0], knnx2[0], nm2[0])
    tgt_nm, tgt_knn_idx, tgt_knn_m, tgt_knn_pts = _finish_partition(
        knn2[1], knnx2[1], nm2[1])

    tgt_corr_idx, src_corr_idx = _run_coarse(tgt_node_f, src_node_f, tgt_nm, src_nm)

    src_ck_idx = src_knn_idx[src_corr_idx]
    tgt_ck_idx = tgt_knn_idx[tgt_corr_idx]
    src_ck_m = src_knn_m[src_corr_idx]
    tgt_ck_m = tgt_knn_m[tgt_corr_idx]
    src_ck_pts = src_knn_pts[src_corr_idx]
    tgt_ck_pts = tgt_knn_pts[tgt_corr_idx]

    src_ck_f = jnp.where(src_ck_m[:, :, None],
                         (src_ck_pts @ W_point) @ W_fine + b_fine, 0.0)
    tgt_ck_f = jnp.where(tgt_ck_m[:, :, None],
                         (tgt_ck_pts @ W_point) @ W_fine + b_fine, 0.0)

    ms = jnp.einsum('bnd,bmd->bnm', tgt_ck_f, src_ck_f) / (W_fine.shape[1] ** 0.5)
    ms, corr_scores, tgt_corr_pts, src_corr_pts = _run_ot(
        ms, tgt_ck_m, src_ck_m, alpha, tgt_ck_pts, src_ck_pts)
    return ms, tgt_corr_pts, src_corr_pts, corr_scores
